# Initial kernel scaffold; baseline (speedup 1.0000x reference)
import jax
import jax.numpy as jnp
from jax.experimental import pallas as pl


def kernel(x, edge_index, edge_weight, W, b):
    raise NotImplementedError("write your pallas kernel here")


# speedup vs baseline: 2.1517x; 2.1517x over previous
import functools

import jax
import jax.numpy as jnp
from jax import lax
from jax.experimental import pallas as pl
from jax.experimental.pallas import tpu as pltpu
from jax.experimental.pallas import tpu_sc as plsc

N = 10000
E = 160000
D = 256
DH = D // 2
NC, NS, L = 2, 16, 16
E_PER_TILE = E // NS
N_SLICE = 624
N_TAIL = N - NS * N_SLICE
BLK = 16
NBLK = E_PER_TILE // BLK


def _sc_spmm_body(x2_hbm, dst_hbm, src_hbm, w_hbm, zeros_hbm, out_hbm,
                  agg_sh, src_v, dst_v, w_v, rows_v, gsem):
    cid = lax.axis_index("c")
    sid = lax.axis_index("s")

    pltpu.sync_copy(zeros_hbm.at[pl.ds(0, N_SLICE)],
                    agg_sh.at[pl.ds(sid * N_SLICE, N_SLICE)])

    @pl.when(sid == NS - 1)
    def _zero_tail():
        pltpu.sync_copy(zeros_hbm.at[pl.ds(0, N_TAIL)],
                        agg_sh.at[pl.ds(NS * N_SLICE, N_TAIL)])

    e0 = sid * E_PER_TILE
    pltpu.sync_copy(src_hbm.at[pl.ds(e0, E_PER_TILE)], src_v)
    pltpu.sync_copy(dst_hbm.at[pl.ds(e0, E_PER_TILE)], dst_v)
    pltpu.sync_copy(w_hbm.at[pl.ds(e0, E_PER_TILE)], w_v)

    plsc.subcore_barrier()

    half_base = cid * N

    def block(b, _):
        ebase = b * BLK
        src16 = src_v[pl.ds(ebase, L)] + half_base
        pltpu.async_copy(x2_hbm.at[src16], rows_v, gsem).wait()
        w16 = w_v[pl.ds(ebase, L)]
        for j in range(BLK):
            wj = w16[j]
            for k in range(DH // L):
                sl = pl.ds(k * L, L)
                rows_v[j, sl] = rows_v[j, sl] * wj
        dst16 = dst_v[pl.ds(ebase, L)]
        pltpu.sync_copy(rows_v, agg_sh.at[dst16], add=True)
        return _

    lax.fori_loop(0, NBLK, block, None)

    plsc.subcore_barrier()

    r0 = sid * N_SLICE
    pltpu.sync_copy(agg_sh.at[pl.ds(r0, N_SLICE)],
                    out_hbm.at[cid, pl.ds(r0, N_SLICE)])

    @pl.when(sid == NS - 1)
    def _stage_tail():
        pltpu.sync_copy(agg_sh.at[pl.ds(NS * N_SLICE, N_TAIL)],
                        out_hbm.at[cid, pl.ds(NS * N_SLICE, N_TAIL)])


@jax.jit
def _sc_spmm(x2, dst, src, w, zeros):
    mesh = plsc.VectorSubcoreMesh(core_axis_name="c", subcore_axis_name="s")
    kern = pl.kernel(
        _sc_spmm_body,
        out_type=jax.ShapeDtypeStruct((NC, N, DH), jnp.float32),
        mesh=mesh,
        scratch_types=[
            pltpu.VMEM_SHARED((N, DH), jnp.float32),
            pltpu.VMEM((E_PER_TILE,), jnp.int32),
            pltpu.VMEM((E_PER_TILE,), jnp.int32),
            pltpu.VMEM((E_PER_TILE,), jnp.float32),
            pltpu.VMEM((BLK, DH), jnp.float32),
            pltpu.SemaphoreType.DMA,
        ],
    )
    return kern(x2, dst, src, w, zeros)


def _tc_linear_body(alo_ref, ahi_ref, wlo_ref, whi_ref, b_ref, o_ref):
    alo = alo_ref[0]
    ahi = ahi_ref[0]
    dn = (((1,), (1,)), ((), ()))
    acc = lax.dot_general(alo, wlo_ref[...], dn,
                          preferred_element_type=jnp.float32)
    acc = acc + lax.dot_general(ahi, whi_ref[...], dn,
                                preferred_element_type=jnp.float32)
    o_ref[...] = jnp.maximum(acc + b_ref[...], 0.0)


@jax.jit
def _tc_linear(agg2, wlo, whi, b2):
    R = 2000
    grid = (N // R,)
    return pl.pallas_call(
        _tc_linear_body,
        grid=grid,
        in_specs=[
            pl.BlockSpec((1, R, DH), lambda i: (0, i, 0)),
            pl.BlockSpec((1, R, DH), lambda i: (1, i, 0)),
            pl.BlockSpec((D, DH), lambda i: (0, 0)),
            pl.BlockSpec((D, DH), lambda i: (0, 0)),
            pl.BlockSpec((1, D), lambda i: (0, 0)),
        ],
        out_specs=pl.BlockSpec((R, D), lambda i: (i, 0)),
        out_shape=jax.ShapeDtypeStruct((N, D), jnp.float32),
    )(agg2, agg2, wlo, whi, b2)


def kernel(x, edge_index, edge_weight, W, b):
    x2 = jnp.concatenate([x[:, :DH], x[:, DH:]], axis=0)
    dst = edge_index[0]
    src = edge_index[1]
    zeros = jnp.zeros((N_SLICE, DH), jnp.float32)
    agg2 = _sc_spmm(x2, dst, src, edge_weight, zeros)
    return _tc_linear(agg2, W[:, :DH], W[:, DH:], b.reshape(1, D))


# speedup vs baseline: 5.3124x; 2.4690x over previous
import functools

import jax
import jax.numpy as jnp
from jax import lax
from jax.experimental import pallas as pl
from jax.experimental.pallas import tpu as pltpu
from jax.experimental.pallas import tpu_sc as plsc

N = 10000
E = 160000
D = 256
DH = D // 2
NC, NS, L = 2, 16, 16
E_PER_TILE = E // NS
N_SLICE = 624
N_TAIL = N - NS * N_SLICE
BLK = 80
SEG = 2000
NSEG = E_PER_TILE // SEG
SBLK = SEG // BLK


def _sc_spmm_body(x2_hbm, dst3_hbm, src_hbm, w_hbm, zeros_hbm, out_hbm,
                  agg_sh, src_v, dst_v, w_v, rows0_v, rows1_v, gsem0, gsem1):
    cid = lax.axis_index("c")
    sid = lax.axis_index("s")

    pltpu.sync_copy(zeros_hbm.at[pl.ds(0, N_SLICE)],
                    agg_sh.at[pl.ds(sid * N_SLICE, N_SLICE)])

    @pl.when(sid == NS - 1)
    def _zero_tail():
        pltpu.sync_copy(zeros_hbm.at[pl.ds(0, N_TAIL)],
                        agg_sh.at[pl.ds(NS * N_SLICE, N_TAIL)])

    half_base = cid * N
    bufs = (rows0_v, rows1_v)
    sems = (gsem0, gsem1)

    def fire(g, buf, sem):
        pltpu.make_async_copy(
            x2_hbm.at[src_v.at[pl.ds(g * BLK, BLK)]], buf, sem).start()

    def wait(buf, sem):
        pltpu.make_async_copy(x2_hbm.at[pl.ds(0, BLK)], buf, sem).wait()

    def process(g, buf):
        for jj in range(BLK // L):
            w16 = w_v[pl.ds(g * BLK + jj * L, L)]
            for j in range(L):
                wj = w16[j]
                r = jj * L + j
                for k in range(DH // L):
                    sl = pl.ds(k * L, L)
                    buf[r, sl] = buf[r, sl] * wj
        pltpu.sync_copy(buf, agg_sh.at[dst_v.at[g]], add=True)

    def segment(seg, _):
        e0 = sid * E_PER_TILE + seg * SEG
        pltpu.sync_copy(src_hbm.at[pl.ds(e0, SEG)], src_v)
        pltpu.sync_copy(dst3_hbm.at[sid, seg], dst_v)
        pltpu.sync_copy(w_hbm.at[pl.ds(e0, SEG)], w_v)

        def adjust(i, _):
            sl = pl.ds(i * L, L)
            src_v[sl] = src_v[sl] + half_base
            return _

        lax.fori_loop(0, SEG // L, adjust, None)

        fire(0, bufs[0], sems[0])

        def step2(t, _):
            g0 = t * 2
            fire(g0 + 1, bufs[1], sems[1])
            wait(bufs[0], sems[0])
            process(g0, bufs[0])
            fire(g0 + 2, bufs[0], sems[0])
            wait(bufs[1], sems[1])
            process(g0 + 1, bufs[1])
            return _

        lax.fori_loop(0, (SBLK - 1) // 2, step2, None)
        wait(bufs[0], sems[0])
        process(SBLK - 1, bufs[0])
        return _

    lax.fori_loop(0, NSEG, segment, None)

    plsc.subcore_barrier()

    r0 = sid * N_SLICE
    pltpu.sync_copy(agg_sh.at[pl.ds(r0, N_SLICE)],
                    out_hbm.at[cid, pl.ds(r0, N_SLICE)])

    @pl.when(sid == NS - 1)
    def _stage_tail():
        pltpu.sync_copy(agg_sh.at[pl.ds(NS * N_SLICE, N_TAIL)],
                        out_hbm.at[cid, pl.ds(NS * N_SLICE, N_TAIL)])


@jax.jit
def _sc_spmm(x2, dst3, src, w, zeros):
    mesh = plsc.VectorSubcoreMesh(core_axis_name="c", subcore_axis_name="s")
    kern = pl.kernel(
        _sc_spmm_body,
        out_type=jax.ShapeDtypeStruct((NC, N, DH), jnp.float32),
        mesh=mesh,
        scratch_types=[
            pltpu.VMEM_SHARED((N, DH), jnp.float32),
            pltpu.VMEM((SEG,), jnp.int32),
            pltpu.VMEM((SBLK, BLK), jnp.int32),
            pltpu.VMEM((SEG,), jnp.float32),
            pltpu.VMEM((BLK, DH), jnp.float32),
            pltpu.VMEM((BLK, DH), jnp.float32),
            pltpu.SemaphoreType.DMA,
            pltpu.SemaphoreType.DMA,
        ],
    )
    return kern(x2, dst3, src, w, zeros)


def _tc_linear_body(alo_ref, ahi_ref, wlo_ref, whi_ref, b_ref, o_ref):
    alo = alo_ref[0]
    ahi = ahi_ref[0]
    dn = (((1,), (1,)), ((), ()))
    acc = lax.dot_general(alo, wlo_ref[...], dn,
                          preferred_element_type=jnp.float32)
    acc = acc + lax.dot_general(ahi, whi_ref[...], dn,
                                preferred_element_type=jnp.float32)
    o_ref[...] = jnp.maximum(acc + b_ref[...], 0.0)


@jax.jit
def _tc_linear(agg2, wlo, whi, b2):
    R = 2000
    grid = (N // R,)
    return pl.pallas_call(
        _tc_linear_body,
        grid=grid,
        in_specs=[
            pl.BlockSpec((1, R, DH), lambda i: (0, i, 0)),
            pl.BlockSpec((1, R, DH), lambda i: (1, i, 0)),
            pl.BlockSpec((D, DH), lambda i: (0, 0)),
            pl.BlockSpec((D, DH), lambda i: (0, 0)),
            pl.BlockSpec((1, D), lambda i: (0, 0)),
        ],
        out_specs=pl.BlockSpec((R, D), lambda i: (i, 0)),
        out_shape=jax.ShapeDtypeStruct((N, D), jnp.float32),
    )(agg2, agg2, wlo, whi, b2)


def kernel(x, edge_index, edge_weight, W, b):
    x2 = jnp.concatenate([x[:, :DH], x[:, DH:]], axis=0)
    dst3 = edge_index[0].reshape(NS, NSEG, SBLK, BLK)
    src = edge_index[1]
    zeros = jnp.zeros((N_SLICE, DH), jnp.float32)
    agg2 = _sc_spmm(x2, dst3, src, edge_weight, zeros)
    return _tc_linear(agg2, W[:, :DH], W[:, DH:], b.reshape(1, D))
